# baseline (speedup 1.0000x reference)
import functools

import jax
import jax.numpy as jnp
from jax import lax
from jax.experimental import pallas as pl
from jax.experimental.pallas import tpu as pltpu
from jax.experimental.pallas import tpu_sc as plsc

MASK_PROP = 0.75


def _index_kernel(num_mask, rl_ref, rs_ref, i_ref, u_ref):
    n = rl_ref.shape[2]
    num_unmask = n - num_mask
    chunk = 48

    v = rl_ref[0, 0, :][None, :]
    rank = jnp.zeros((1, n), jnp.float32)
    for c in range(0, n, chunk):
        vj = rs_ref[0, c : c + chunk, :]
        rank = rank + jnp.sum((vj < v).astype(jnp.float32), axis=0, keepdims=True)
    ranki = rank.astype(jnp.int32)

    col = jax.lax.broadcasted_iota(jnp.int32, (chunk, n), 1)
    for c in range(0, num_mask, chunk):
        mk = jax.lax.broadcasted_iota(jnp.int32, (chunk, n), 0) + c
        sel = ranki == mk
        i_ref[0, 0, c : c + chunk] = jnp.sum(jnp.where(sel, col, 0), axis=1)
    base = pl.program_id(0) * n
    for c in range(0, num_unmask, chunk):
        uk = jax.lax.broadcasted_iota(jnp.int32, (chunk, n), 0) + num_mask + c
        sel = ranki == uk
        u_ref[0, 0, c : c + chunk] = jnp.sum(jnp.where(sel, col, 0), axis=1) + base


def _make_sc_encoder(b, n, e, num_mask):
    num_unmask = n - num_mask
    info = plsc.get_sparse_core_info()
    nc, ns = info.num_cores, info.num_subcores
    nw = nc * ns
    bpw = b // nw
    gchunk = 48
    fchunk = 48

    @functools.partial(
        pl.kernel,
        mesh=plsc.VectorSubcoreMesh(core_axis_name="c", subcore_axis_name="s"),
        out_type=jax.ShapeDtypeStruct((b * n, e), jnp.float32),
        scratch_types=[
            pltpu.VMEM((gchunk,), jnp.int32),
            pltpu.VMEM((gchunk, e), jnp.float32),
            pltpu.VMEM((fchunk, e), jnp.float32),
            pltpu.VMEM((fchunk,), jnp.int32),
            pltpu.SemaphoreType.DMA,
        ],
    )
    def sc_encode(patches_hbm, uidx_hbm, mask_hbm, out_hbm, idx_v, rows_v, fill_v, zidx_v, sem):
        wid = lax.axis_index("s") * nc + lax.axis_index("c")
        for c in range(0, fchunk, 16):
            zidx_v[pl.ds(c, 16)] = jnp.zeros((16,), jnp.int32)
        pltpu.async_copy(mask_hbm.at[zidx_v], fill_v, sem).wait()
        for bb in range(bpw):
            bat = wid * bpw + bb
            for c in range(0, num_unmask, gchunk):
                pltpu.sync_copy(
                    uidx_hbm.at[pl.ds(bat * num_unmask + c, gchunk)], idx_v
                )
                pltpu.async_copy(patches_hbm.at[idx_v], rows_v, sem).wait()
                pltpu.sync_copy(rows_v, out_hbm.at[pl.ds(bat * n + c, gchunk)])
            for c in range(0, num_mask, fchunk):
                pltpu.sync_copy(
                    fill_v, out_hbm.at[pl.ds(bat * n + num_unmask + c, fchunk)]
                )

    return sc_encode


def kernel(patches, mask_token):
    b, n, e = patches.shape
    num_mask = -(-3 * n // 4)
    num_unmask = n - num_mask

    rkey = jax.random.key(42)
    rand_vals = jax.random.uniform(rkey, (b, n), dtype=jnp.float32)
    rand_lane = rand_vals.reshape(b, 1, n)
    rand_sub = rand_vals.reshape(b, n, 1)

    midx3, uidx3 = pl.pallas_call(
        functools.partial(_index_kernel, num_mask),
        grid=(b,),
        in_specs=[
            pl.BlockSpec((1, 1, n), lambda i: (i, 0, 0)),
            pl.BlockSpec((1, n, 1), lambda i: (i, 0, 0)),
        ],
        out_specs=[
            pl.BlockSpec((1, 1, num_mask), lambda i: (i, 0, 0)),
            pl.BlockSpec((1, 1, num_unmask), lambda i: (i, 0, 0)),
        ],
        out_shape=[
            jax.ShapeDtypeStruct((b, 1, num_mask), jnp.int32),
            jax.ShapeDtypeStruct((b, 1, num_unmask), jnp.int32),
        ],
    )(rand_lane, rand_sub)

    sc_encode = _make_sc_encoder(b, n, e, num_mask)
    out_flat = sc_encode(
        patches.reshape(b * n, e),
        uidx3.reshape(b * num_unmask),
        mask_token,
    )
    return out_flat.reshape(b, n, e), midx3.reshape(b, num_mask)


# speedup vs baseline: 1.0060x; 1.0060x over previous
import functools

import jax
import jax.numpy as jnp
from jax import lax
from jax.experimental import pallas as pl
from jax.experimental.pallas import tpu as pltpu
from jax.experimental.pallas import tpu_sc as plsc

MASK_PROP = 0.75


def _index_kernel(num_mask, rl_ref, rs_ref, i_ref, u_ref):
    n = rl_ref.shape[2]
    num_unmask = n - num_mask
    chunk = 48

    v = rl_ref[0, 0, :][None, :]
    rank = jnp.zeros((1, n), jnp.float32)
    for c in range(0, n, chunk):
        vj = rs_ref[0, c : c + chunk, :]
        rank = rank + jnp.sum((vj < v).astype(jnp.float32), axis=0, keepdims=True)
    ranki = rank.astype(jnp.int32)

    col = jax.lax.broadcasted_iota(jnp.int32, (chunk, n), 1)
    for c in range(0, num_mask, chunk):
        mk = jax.lax.broadcasted_iota(jnp.int32, (chunk, n), 0) + c
        sel = ranki == mk
        i_ref[0, 0, c : c + chunk] = jnp.sum(jnp.where(sel, col, 0), axis=1)
    base = pl.program_id(0) * n
    for c in range(0, num_unmask, chunk):
        uk = jax.lax.broadcasted_iota(jnp.int32, (chunk, n), 0) + num_mask + c
        sel = ranki == uk
        u_ref[0, 0, c : c + chunk] = jnp.sum(jnp.where(sel, col, 0), axis=1) + base


def _make_sc_encoder(b, n, e, num_mask):
    num_unmask = n - num_mask
    info = plsc.get_sparse_core_info()
    nc, ns = info.num_cores, info.num_subcores
    nw = nc * ns
    bpw = b // nw
    gchunk = 48
    fchunk = 48

    n_g = (num_unmask // gchunk) * bpw

    @functools.partial(
        pl.kernel,
        mesh=plsc.VectorSubcoreMesh(core_axis_name="c", subcore_axis_name="s"),
        out_type=jax.ShapeDtypeStruct((b * n, e), jnp.float32),
        scratch_types=[
            pltpu.VMEM((bpw * num_unmask,), jnp.int32),
            pltpu.VMEM((gchunk, e), jnp.float32),
            pltpu.VMEM((gchunk, e), jnp.float32),
            pltpu.VMEM((fchunk, e), jnp.float32),
            pltpu.VMEM((fchunk,), jnp.int32),
            pltpu.SemaphoreType.DMA,
            pltpu.SemaphoreType.DMA,
            pltpu.SemaphoreType.DMA,
        ],
    )
    def sc_encode(
        patches_hbm, uidx_hbm, mask_hbm, out_hbm,
        idx_v, rows_a, rows_b, fill_v, zidx_v, sem_a, sem_b, sem_f,
    ):
        wid = lax.axis_index("s") * nc + lax.axis_index("c")
        for c in range(0, fchunk, 16):
            zidx_v[pl.ds(c, 16)] = jnp.zeros((16,), jnp.int32)
        mcp = pltpu.async_copy(mask_hbm.at[zidx_v], fill_v, sem_f)
        pltpu.sync_copy(
            uidx_hbm.at[pl.ds(wid * bpw * num_unmask, bpw * num_unmask)], idx_v
        )
        mcp.wait()
        fills = []
        for bb in range(bpw):
            bat = wid * bpw + bb
            for c in range(0, num_mask, fchunk):
                fills.append(
                    pltpu.async_copy(
                        fill_v, out_hbm.at[pl.ds(bat * n + num_unmask + c, fchunk)], sem_f
                    )
                )
        bufs = (rows_a, rows_b)
        sems = (sem_a, sem_b)

        def out_off(g):
            bat = wid * bpw + g // (num_unmask // gchunk)
            return bat * n + (g % (num_unmask // gchunk)) * gchunk

        def start(g):
            return pltpu.async_copy(
                patches_hbm.at[idx_v.at[pl.ds(g * gchunk, gchunk)]],
                bufs[g % 2], sems[g % 2],
            )

        cps = [start(0), start(1)]
        for g in range(n_g):
            cps[g % 2].wait()
            pltpu.sync_copy(bufs[g % 2], out_hbm.at[pl.ds(out_off(g), gchunk)])
            if g + 2 < n_g:
                cps[g % 2] = start(g + 2)
        for f in fills:
            f.wait()

    return sc_encode


def kernel(patches, mask_token):
    b, n, e = patches.shape
    num_mask = -(-3 * n // 4)
    num_unmask = n - num_mask

    rkey = jax.random.key(42)
    rand_vals = jax.random.uniform(rkey, (b, n), dtype=jnp.float32)
    rand_lane = rand_vals.reshape(b, 1, n)
    rand_sub = rand_vals.reshape(b, n, 1)

    midx3, uidx3 = pl.pallas_call(
        functools.partial(_index_kernel, num_mask),
        grid=(b,),
        in_specs=[
            pl.BlockSpec((1, 1, n), lambda i: (i, 0, 0)),
            pl.BlockSpec((1, n, 1), lambda i: (i, 0, 0)),
        ],
        out_specs=[
            pl.BlockSpec((1, 1, num_mask), lambda i: (i, 0, 0)),
            pl.BlockSpec((1, 1, num_unmask), lambda i: (i, 0, 0)),
        ],
        out_shape=[
            jax.ShapeDtypeStruct((b, 1, num_mask), jnp.int32),
            jax.ShapeDtypeStruct((b, 1, num_unmask), jnp.int32),
        ],
    )(rand_lane, rand_sub)

    sc_encode = _make_sc_encoder(b, n, e, num_mask)
    out_flat = sc_encode(
        patches.reshape(b * n, e),
        uidx3.reshape(b * num_unmask),
        mask_token,
    )
    return out_flat.reshape(b, n, e), midx3.reshape(b, num_mask)


# speedup vs baseline: 2.3403x; 2.3263x over previous
import functools

import jax
import jax.numpy as jnp
from jax.experimental import pallas as pl

MASK_PROP = 0.75


def _mask_encode_kernel(num_mask, p_ref, rl_ref, rc_ref, m_ref, e_ref, i_ref):
    n = p_ref.shape[1]
    num_unmask = n - num_mask
    chunk = rc_ref.shape[1]
    cols = rc_ref.shape[2]

    v = rl_ref[0, 0, :][None, :]
    rank = jnp.zeros((1, n), jnp.float32)
    for t in range(cols):
        vj = rc_ref[0, :, t : t + 1]
        rank = rank + jnp.sum((vj < v).astype(jnp.float32), axis=0, keepdims=True)
    ranki = rank.astype(jnp.int32)

    kk = jax.lax.broadcasted_iota(jnp.int32, (num_unmask, n), 0) + num_mask
    onehot = (ranki == kk).astype(jnp.float32)
    e_ref[0, :num_unmask, :] = jnp.dot(
        onehot, p_ref[0], preferred_element_type=jnp.float32
    )
    e_ref[0, num_unmask:, :] = jnp.broadcast_to(
        m_ref[0, :], (num_mask, e_ref.shape[2])
    )

    col = jax.lax.broadcasted_iota(jnp.int32, (chunk, n), 1)
    for c in range(0, num_mask, chunk):
        mk = jax.lax.broadcasted_iota(jnp.int32, (chunk, n), 0) + c
        sel = ranki == mk
        i_ref[0, 0, c : c + chunk] = jnp.sum(jnp.where(sel, col, 0), axis=1)


def kernel(patches, mask_token):
    b, n, e = patches.shape
    num_mask = -(-3 * n // 4)

    rkey = jax.random.key(42)
    rand_vals = jax.random.uniform(rkey, (b, n), dtype=jnp.float32)
    rand_lane = rand_vals.reshape(b, 1, n)
    rand_cols = rand_vals.reshape(b, n // 12, 12)

    enc, idx3 = pl.pallas_call(
        functools.partial(_mask_encode_kernel, num_mask),
        grid=(b,),
        in_specs=[
            pl.BlockSpec((1, n, e), lambda i: (i, 0, 0)),
            pl.BlockSpec((1, 1, n), lambda i: (i, 0, 0)),
            pl.BlockSpec((1, n // 12, 12), lambda i: (i, 0, 0)),
            pl.BlockSpec((1, e), lambda i: (0, 0)),
        ],
        out_specs=[
            pl.BlockSpec((1, n, e), lambda i: (i, 0, 0)),
            pl.BlockSpec((1, 1, num_mask), lambda i: (i, 0, 0)),
        ],
        out_shape=[
            jax.ShapeDtypeStruct((b, n, e), jnp.float32),
            jax.ShapeDtypeStruct((b, 1, num_mask), jnp.int32),
        ],
    )(patches, rand_lane, rand_cols, mask_token)
    return enc, idx3.reshape(b, num_mask)


# speedup vs baseline: 2.7441x; 1.1726x over previous
import functools

import jax
import jax.numpy as jnp
from jax.experimental import pallas as pl

MASK_PROP = 0.75


def _mask_encode_kernel(num_mask, p_ref, rl_ref, rc_ref, m_ref, e_ref, i_ref):
    n = p_ref.shape[1]
    num_unmask = n - num_mask
    chunk = rc_ref.shape[1]
    cols = rc_ref.shape[2]

    kk = jax.lax.broadcasted_iota(jnp.int32, (num_unmask, n), 0) + num_mask
    col = jax.lax.broadcasted_iota(jnp.int32, (chunk, n), 1)
    for bb in range(p_ref.shape[0]):
        v = rl_ref[bb, 0, :][None, :]
        rank = jnp.zeros((1, n), jnp.float32)
        for t in range(cols):
            vj = rc_ref[bb, :, t : t + 1]
            rank = rank + jnp.sum(
                (vj < v).astype(jnp.float32), axis=0, keepdims=True
            )
        ranki = rank.astype(jnp.int32)

        onehot = (ranki == kk).astype(jnp.float32)
        e_ref[bb, :num_unmask, :] = jnp.dot(
            onehot, p_ref[bb], preferred_element_type=jnp.float32
        )
        e_ref[bb, num_unmask:, :] = jnp.broadcast_to(
            m_ref[0, :], (num_mask, e_ref.shape[2])
        )

        for c in range(0, num_mask, chunk):
            mk = jax.lax.broadcasted_iota(jnp.int32, (chunk, n), 0) + c
            sel = ranki == mk
            i_ref[bb, 0, c : c + chunk] = jnp.sum(jnp.where(sel, col, 0), axis=1)


def kernel(patches, mask_token):
    b, n, e = patches.shape
    num_mask = -(-3 * n // 4)

    rkey = jax.random.key(42)
    rand_vals = jax.random.uniform(rkey, (b, n), dtype=jnp.float32)
    rand_lane = rand_vals.reshape(b, 1, n)
    rand_cols = rand_vals.reshape(b, n // 12, 12)

    bb = 2
    enc, idx3 = pl.pallas_call(
        functools.partial(_mask_encode_kernel, num_mask),
        grid=(b // bb,),
        in_specs=[
            pl.BlockSpec((bb, n, e), lambda i: (i, 0, 0)),
            pl.BlockSpec((bb, 1, n), lambda i: (i, 0, 0)),
            pl.BlockSpec((bb, n // 12, 12), lambda i: (i, 0, 0)),
            pl.BlockSpec((1, e), lambda i: (0, 0)),
        ],
        out_specs=[
            pl.BlockSpec((bb, n, e), lambda i: (i, 0, 0)),
            pl.BlockSpec((bb, 1, num_mask), lambda i: (i, 0, 0)),
        ],
        out_shape=[
            jax.ShapeDtypeStruct((b, n, e), jnp.float32),
            jax.ShapeDtypeStruct((b, 1, num_mask), jnp.int32),
        ],
    )(patches, rand_lane, rand_cols, mask_token)
    return enc, idx3.reshape(b, num_mask)


# speedup vs baseline: 2.9245x; 1.0657x over previous
import functools

import jax
import jax.numpy as jnp
from jax.experimental import pallas as pl

MASK_PROP = 0.75


def _mask_encode_kernel(num_mask, p_ref, rl_ref, rc_ref, m_ref, e_ref, i_ref):
    n = p_ref.shape[1]
    num_unmask = n - num_mask
    chunk = rc_ref.shape[1]
    cols = rc_ref.shape[2]

    kk = jax.lax.broadcasted_iota(jnp.int32, (num_unmask, n), 0) + num_mask
    col = jax.lax.broadcasted_iota(jnp.int32, (chunk, n), 1)
    for bb in range(p_ref.shape[0]):
        v = rl_ref[bb, 0, :][None, :]
        rank = jnp.zeros((1, n), jnp.float32)
        for t in range(cols):
            vj = rc_ref[bb, :, t : t + 1]
            rank = rank + jnp.sum(
                (vj < v).astype(jnp.float32), axis=0, keepdims=True
            )
        ranki = rank.astype(jnp.int32)

        onehot = (ranki == kk).astype(jnp.float32)
        e_ref[bb, :num_unmask, :] = jnp.dot(
            onehot, p_ref[bb], preferred_element_type=jnp.float32
        )
        e_ref[bb, num_unmask:, :] = jnp.broadcast_to(
            m_ref[0, :], (num_mask, e_ref.shape[2])
        )

        for c in range(0, num_mask, chunk):
            mk = jax.lax.broadcasted_iota(jnp.int32, (chunk, n), 0) + c
            sel = ranki == mk
            i_ref[bb, 0, c : c + chunk] = jnp.sum(jnp.where(sel, col, 0), axis=1)


def kernel(patches, mask_token):
    b, n, e = patches.shape
    num_mask = -(-3 * n // 4)

    rkey = jax.random.key(42)
    rand_vals = jax.random.uniform(rkey, (b, n), dtype=jnp.float32)
    rand_lane = rand_vals.reshape(b, 1, n)
    rand_cols = rand_vals.reshape(b, n // 12, 12)

    bb = 4
    enc, idx3 = pl.pallas_call(
        functools.partial(_mask_encode_kernel, num_mask),
        grid=(b // bb,),
        in_specs=[
            pl.BlockSpec((bb, n, e), lambda i: (i, 0, 0)),
            pl.BlockSpec((bb, 1, n), lambda i: (i, 0, 0)),
            pl.BlockSpec((bb, n // 12, 12), lambda i: (i, 0, 0)),
            pl.BlockSpec((1, e), lambda i: (0, 0)),
        ],
        out_specs=[
            pl.BlockSpec((bb, n, e), lambda i: (i, 0, 0)),
            pl.BlockSpec((bb, 1, num_mask), lambda i: (i, 0, 0)),
        ],
        out_shape=[
            jax.ShapeDtypeStruct((b, n, e), jnp.float32),
            jax.ShapeDtypeStruct((b, 1, num_mask), jnp.int32),
        ],
    )(patches, rand_lane, rand_cols, mask_token)
    return enc, idx3.reshape(b, num_mask)
